# baseline (speedup 1.0000x reference)
import functools

import jax
import jax.numpy as jnp
from jax.experimental import pallas as pl
from jax.experimental.pallas import tpu as pltpu

BM = 200


def _gcn_kernel(a_ref, x_ref, w_ref, b_ref, out_ref, support_ref):
    i = pl.program_id(0)

    @pl.when(i == 0)
    def _():
        support_ref[...] = jnp.dot(
            x_ref[...], w_ref[...], preferred_element_type=jnp.float32
        )

    out_ref[...] = (
        jnp.dot(
            a_ref[...].astype(jnp.bfloat16),
            support_ref[...].astype(jnp.bfloat16),
            preferred_element_type=jnp.float32,
        )
        + b_ref[...]
    )


@jax.jit
def kernel(adjacency, input_feature, weight, bias):
    n, d_in = input_feature.shape
    d_out = weight.shape[1]
    bias2d = bias.reshape(1, d_out)
    out = pl.pallas_call(
        _gcn_kernel,
        grid=(n // BM,),
        in_specs=[
            pl.BlockSpec((BM, n), lambda i: (i, 0)),
            pl.BlockSpec((n, d_in), lambda i: (0, 0)),
            pl.BlockSpec((d_in, d_out), lambda i: (0, 0)),
            pl.BlockSpec((1, d_out), lambda i: (0, 0)),
        ],
        out_specs=pl.BlockSpec((BM, d_out), lambda i: (i, 0)),
        out_shape=jax.ShapeDtypeStruct((n, d_out), jnp.float32),
        scratch_shapes=[pltpu.VMEM((n, d_out), jnp.float32)],
    )(adjacency, input_feature, weight, bias2d)
    return out


# speedup vs baseline: 1.0049x; 1.0049x over previous
import functools

import jax
import jax.numpy as jnp
from jax.experimental import pallas as pl
from jax.experimental.pallas import tpu as pltpu

BM = 200


def _gcn_kernel(a0_ref, a1_ref, x_ref, w_ref, b_ref, out_ref, support_ref):
    i = pl.program_id(0)

    @pl.when(i == 0)
    def _():
        support_ref[...] = jnp.dot(
            x_ref[...], w_ref[...], preferred_element_type=jnp.float32
        )

    s = support_ref[...].astype(jnp.bfloat16)
    b = b_ref[...]
    out_ref[:BM, :] = (
        jnp.dot(a0_ref[...].astype(jnp.bfloat16), s,
                preferred_element_type=jnp.float32) + b
    )
    out_ref[BM:, :] = (
        jnp.dot(a1_ref[...].astype(jnp.bfloat16), s,
                preferred_element_type=jnp.float32) + b
    )


@jax.jit
def kernel(adjacency, input_feature, weight, bias):
    n, d_in = input_feature.shape
    d_out = weight.shape[1]
    bias2d = bias.reshape(1, d_out)
    out = pl.pallas_call(
        _gcn_kernel,
        grid=(n // (2 * BM),),
        in_specs=[
            pl.BlockSpec((BM, n), lambda i: (2 * i, 0)),
            pl.BlockSpec((BM, n), lambda i: (2 * i + 1, 0)),
            pl.BlockSpec((n, d_in), lambda i: (0, 0)),
            pl.BlockSpec((d_in, d_out), lambda i: (0, 0)),
            pl.BlockSpec((1, d_out), lambda i: (0, 0)),
        ],
        out_specs=pl.BlockSpec((2 * BM, d_out), lambda i: (i, 0)),
        out_shape=jax.ShapeDtypeStruct((n, d_out), jnp.float32),
        scratch_shapes=[pltpu.VMEM((n, d_out), jnp.float32)],
    )(adjacency, adjacency, input_feature, weight, bias2d)
    return out


# speedup vs baseline: 1.0135x; 1.0086x over previous
import jax
import jax.numpy as jnp
from jax.experimental import pallas as pl
from jax.experimental.pallas import tpu as pltpu

BM = 400


def _gcn_kernel(a_ref, x_ref, w_ref, b_ref, out_ref, support_ref):
    i = pl.program_id(0)

    @pl.when(i == 0)
    def _():
        support_ref[...] = jnp.dot(
            x_ref[...], w_ref[...], preferred_element_type=jnp.float32
        )

    out_ref[...] = (
        jnp.dot(
            a_ref[...].astype(jnp.bfloat16),
            support_ref[...].astype(jnp.bfloat16),
            preferred_element_type=jnp.float32,
        )
        + b_ref[...]
    )


@jax.jit
def kernel(adjacency, input_feature, weight, bias):
    n, d_in = input_feature.shape
    d_out = weight.shape[1]
    bias2d = bias.reshape(1, d_out)
    out = pl.pallas_call(
        _gcn_kernel,
        grid=(n // BM,),
        in_specs=[
            pl.BlockSpec((BM, n), lambda i: (i, 0)),
            pl.BlockSpec((n, d_in), lambda i: (0, 0)),
            pl.BlockSpec((d_in, d_out), lambda i: (0, 0)),
            pl.BlockSpec((1, d_out), lambda i: (0, 0)),
        ],
        out_specs=pl.BlockSpec((BM, d_out), lambda i: (i, 0)),
        out_shape=jax.ShapeDtypeStruct((n, d_out), jnp.float32),
        scratch_shapes=[pltpu.VMEM((n, d_out), jnp.float32)],
    )(adjacency, input_feature, weight, bias2d)
    return out
